# baseline (speedup 1.0000x reference)
import functools

import jax
import jax.numpy as jnp
from jax import lax
from jax.experimental import pallas as pl
from jax.experimental.pallas import tpu as pltpu
from jax.experimental.pallas import tpu_sc as plsc

_D = 1000
_N = 51200
_NW = 32
_RPW = _N // _NW
_CHUNK = 40
_NCHUNK = _RPW // _CHUNK


@functools.partial(
    pl.kernel,
    out_type=jax.ShapeDtypeStruct((_N, _D), jnp.float32),
    mesh=plsc.VectorSubcoreMesh(core_axis_name="c", subcore_axis_name="s"),
    compiler_params=pltpu.CompilerParams(use_tc_tiling_on_sc=False),
    scratch_types=[
        pltpu.VMEM((_RPW,), jnp.int32),
        pltpu.VMEM((_CHUNK, _D), jnp.float32),
        pltpu.SemaphoreType.DMA,
    ],
)
def _sc_gather(table_hbm, idx_hbm, out_hbm, idx_v, buf, gsem):
    wid = lax.axis_index("s") * 2 + lax.axis_index("c")
    base = wid * _RPW
    pltpu.sync_copy(idx_hbm.at[pl.ds(base, _RPW)], idx_v)

    def body(c, carry):
        off = c * _CHUNK
        pltpu.async_copy(
            table_hbm.at[idx_v.at[pl.ds(off, _CHUNK)]], buf, gsem
        ).wait()
        pltpu.sync_copy(buf, out_hbm.at[pl.ds(base + off, _CHUNK)])
        return carry

    lax.fori_loop(0, _NCHUNK, body, 0)


def kernel(idx, token_embedding_table):
    out = _sc_gather(token_embedding_table, idx.reshape(-1))
    return out.reshape(idx.shape + (_D,))


# speedup vs baseline: 1.0377x; 1.0377x over previous
import functools

import jax
import jax.numpy as jnp
from jax import lax
from jax.experimental import pallas as pl
from jax.experimental.pallas import tpu as pltpu
from jax.experimental.pallas import tpu_sc as plsc

_D = 1000
_N = 51200
_NW = 32
_RPW = _N // _NW
_CHUNK = 40
_NCHUNK = _RPW // _CHUNK


@functools.partial(
    pl.kernel,
    out_type=jax.ShapeDtypeStruct((_N, _D), jnp.float32),
    mesh=plsc.VectorSubcoreMesh(core_axis_name="c", subcore_axis_name="s"),
    compiler_params=pltpu.CompilerParams(use_tc_tiling_on_sc=False),
    scratch_types=[
        pltpu.VMEM((_RPW,), jnp.int32),
        pltpu.VMEM((2, _CHUNK, _D), jnp.float32),
        pltpu.SemaphoreType.DMA,
        pltpu.SemaphoreType.DMA,
        pltpu.SemaphoreType.DMA,
        pltpu.SemaphoreType.DMA,
    ],
)
def _sc_gather(table_hbm, idx_hbm, out_hbm, idx_v, buf, gs0, gs1, ws0, ws1):
    wid = lax.axis_index("s") * 2 + lax.axis_index("c")
    base = wid * _RPW
    pltpu.sync_copy(idx_hbm.at[pl.ds(base, _RPW)], idx_v)

    gsem = (gs0, gs1)
    wsem = (ws0, ws1)

    def gstart(c, b):
        pltpu.async_copy(
            table_hbm.at[idx_v.at[pl.ds(c * _CHUNK, _CHUNK)]],
            buf.at[b], gsem[b])

    def gwait(b):
        pltpu.make_async_copy(
            table_hbm.at[idx_v.at[pl.ds(0, _CHUNK)]],
            buf.at[b], gsem[b]).wait()

    def wstart(c, b):
        pltpu.async_copy(
            buf.at[b], out_hbm.at[pl.ds(base + c * _CHUNK, _CHUNK)], wsem[b])

    def wwait(b):
        pltpu.make_async_copy(
            buf.at[b], out_hbm.at[pl.ds(base, _CHUNK)], wsem[b]).wait()

    gstart(0, 0)

    def body(i, carry):
        c0 = 2 * i

        @pl.when(i > 0)
        def _():
            wwait(1)
        gstart(c0 + 1, 1)
        gwait(0)
        wstart(c0, 0)
        wwait(0)

        @pl.when(i < _NCHUNK // 2 - 1)
        def _():
            gstart(c0 + 2, 0)
        gwait(1)
        wstart(c0 + 1, 1)
        return carry

    lax.fori_loop(0, _NCHUNK // 2, body, 0)
    wwait(1)


def kernel(idx, token_embedding_table):
    out = _sc_gather(token_embedding_table, idx.reshape(-1))
    return out.reshape(idx.shape + (_D,))


# speedup vs baseline: 1.1484x; 1.1067x over previous
import functools

import jax
import jax.numpy as jnp
from jax import lax
from jax.experimental import pallas as pl
from jax.experimental.pallas import tpu as pltpu
from jax.experimental.pallas import tpu_sc as plsc

_D = 1000
_N = 51200
_NW = 32
_RPW = _N // _NW
_CHUNK = 32
_NCHUNK = _RPW // _CHUNK


@functools.partial(
    pl.kernel,
    out_type=jax.ShapeDtypeStruct((_N, _D), jnp.float32),
    mesh=plsc.VectorSubcoreMesh(core_axis_name="c", subcore_axis_name="s"),
    compiler_params=pltpu.CompilerParams(use_tc_tiling_on_sc=False),
    scratch_types=[
        pltpu.VMEM((_RPW,), jnp.int32),
        pltpu.VMEM((2, _CHUNK, _D), jnp.float32),
        pltpu.VMEM_SHARED((1000, _D), jnp.float32),
        pltpu.SemaphoreType.DMA,
        pltpu.SemaphoreType.DMA,
        pltpu.SemaphoreType.DMA,
        pltpu.SemaphoreType.DMA,
    ],
)
def _sc_gather(table_hbm, idx_hbm, out_hbm, idx_v, buf, tab_sp, gs0, gs1, ws0, ws1):
    sid = lax.axis_index("s")
    wid = sid * 2 + lax.axis_index("c")
    base = wid * _RPW

    @pl.when(sid == 0)
    def _():
        pltpu.sync_copy(table_hbm, tab_sp)

    pltpu.sync_copy(idx_hbm.at[pl.ds(base, _RPW)], idx_v)
    plsc.subcore_barrier()

    gsem = (gs0, gs1)
    wsem = (ws0, ws1)

    def gstart(c, b):
        pltpu.async_copy(
            tab_sp.at[idx_v.at[pl.ds(c * _CHUNK, _CHUNK)]],
            buf.at[b], gsem[b])

    def gwait(b):
        pltpu.make_async_copy(
            tab_sp.at[idx_v.at[pl.ds(0, _CHUNK)]],
            buf.at[b], gsem[b]).wait()

    def wstart(c, b):
        pltpu.async_copy(
            buf.at[b], out_hbm.at[pl.ds(base + c * _CHUNK, _CHUNK)], wsem[b])

    def wwait(b):
        pltpu.make_async_copy(
            buf.at[b], out_hbm.at[pl.ds(base, _CHUNK)], wsem[b]).wait()

    gstart(0, 0)

    def body(i, carry):
        c0 = 2 * i

        @pl.when(i > 0)
        def _():
            wwait(1)
        gstart(c0 + 1, 1)
        gwait(0)
        wstart(c0, 0)
        wwait(0)

        @pl.when(i < _NCHUNK // 2 - 1)
        def _():
            gstart(c0 + 2, 0)
        gwait(1)
        wstart(c0 + 1, 1)
        return carry

    lax.fori_loop(0, _NCHUNK // 2, body, 0)
    wwait(1)


def kernel(idx, token_embedding_table):
    out = _sc_gather(token_embedding_table, idx.reshape(-1))
    return out.reshape(idx.shape + (_D,))
